# baseline (speedup 1.0000x reference)
import functools

import jax
import jax.numpy as jnp
from jax import lax
from jax.experimental import pallas as pl
from jax.experimental.pallas import tpu as pltpu
from jax.experimental.pallas import tpu_sc as plsc

_NUM_USERS = 500000
_EMB = 64
_B = 16384
_NC = 2
_NS = 16
_NW = _NC * _NS
_BPW = _B // _NW
_L = 16


def _body(users_hbm, items_hbm, neg_hbm, table_hbm,
          u_out, v_out, n_out,
          idx_u, idx_i, idx_n, rows_u, rows_i, rows_n,
          sem_u, sem_i, sem_n, sem_o):
    wid = lax.axis_index("s") * _NC + lax.axis_index("c")
    base = wid * _BPW

    pltpu.sync_copy(users_hbm.at[pl.ds(base, _BPW)], idx_u)
    cu = pltpu.async_copy(table_hbm.at[idx_u], rows_u, sem_u)

    pltpu.sync_copy(items_hbm.at[pl.ds(base, _BPW)], idx_i)
    for j in range(_BPW // _L):
        s = pl.ds(j * _L, _L)
        idx_i[s] = idx_i[s] + _NUM_USERS
    ci = pltpu.async_copy(table_hbm.at[idx_i], rows_i, sem_i)

    pltpu.sync_copy(neg_hbm.at[pl.ds(base, _BPW)], idx_n)
    for j in range(_BPW // _L):
        s = pl.ds(j * _L, _L)
        idx_n[s] = idx_n[s] + _NUM_USERS
    cn = pltpu.async_copy(table_hbm.at[idx_n], rows_n, sem_n)

    cu.wait()
    ou = pltpu.async_copy(rows_u, u_out.at[pl.ds(base, _BPW)], sem_o)
    ci.wait()
    oi = pltpu.async_copy(rows_i, v_out.at[pl.ds(base, _BPW)], sem_o)
    cn.wait()
    on = pltpu.async_copy(rows_n, n_out.at[pl.ds(base, _BPW)], sem_o)
    ou.wait()
    oi.wait()
    on.wait()


_gather = functools.partial(
    pl.kernel,
    mesh=plsc.VectorSubcoreMesh(core_axis_name="c", subcore_axis_name="s"),
    compiler_params=pltpu.CompilerParams(use_tc_tiling_on_sc=False),
    out_type=[jax.ShapeDtypeStruct((_B, _EMB), jnp.float32)] * 3,
    scratch_types=[
        pltpu.VMEM((_BPW,), jnp.int32),
        pltpu.VMEM((_BPW,), jnp.int32),
        pltpu.VMEM((_BPW,), jnp.int32),
        pltpu.VMEM((_BPW, _EMB), jnp.float32),
        pltpu.VMEM((_BPW, _EMB), jnp.float32),
        pltpu.VMEM((_BPW, _EMB), jnp.float32),
        pltpu.SemaphoreType.DMA,
        pltpu.SemaphoreType.DMA,
        pltpu.SemaphoreType.DMA,
        pltpu.SemaphoreType.DMA,
    ],
)(_body)


def kernel(users, items, neg_items, U_and_V):
    u, v, n = _gather(users.astype(jnp.int32), items.astype(jnp.int32),
                      neg_items.astype(jnp.int32), U_and_V)
    return (u, v, n)
